# baseline (speedup 1.0000x reference)
import functools

import jax
import jax.numpy as jnp
from jax.experimental import pallas as pl
from jax.experimental.pallas import tpu as pltpu
from jax.experimental.pallas import tpu_sc as plsc

C = 96
H = W = 224
WL = 256
CH = 8192
NCJ = 8
LTOT = CH * NCJ
RB = 32
NCO = 7
E = 8
TOPK = 2
HALO = WL + 1
SHIFTS = tuple(dy * WL + dx for dy in range(3) for dx in range(3))


def _interior_mask(start, length, dtype):
    g = start + jax.lax.broadcasted_iota(jnp.int32, (1, length), 1)
    g = jnp.maximum(g, 0)
    row = g // WL
    col = g - row * WL
    return ((row >= 1) & (row <= H) & (col < W)).astype(dtype)


def _pre_kernel(ph_ref, xm_ref, xc_ref, xp_ref, aw_ref, bw_ref, dww_ref, al_ref,
                rw_ref, rb_ref, fx_ref, sc_ref,
                gap_ref, r1_ref, r2_ref):
    b = pl.program_id(0)
    jj = pl.program_id(1)
    ext = jnp.concatenate(
        [xm_ref[0, :, CH - HALO:], xc_ref[0], xp_ref[0, :, :HALO]], axis=1)
    xb_ext = jnp.dot(bw_ref[...], ext, preferred_element_type=jnp.float32)
    lane = jax.lax.broadcasted_iota(jnp.int32, (1, CH + 2 * HALO), 1)
    xb_ext = jnp.where((jj == 0) & (lane == HALO - 1), 0.0, xb_ext)
    r1_ref[...] = xb_ext[:, 1:1 + CH + 2 * WL]
    r2_ref[...] = xb_ext[:, 2:2 + CH + 2 * WL]
    y = None
    for dy in range(3):
        for dx, src in ((0, None), (1, r1_ref), (2, r2_ref)):
            sl = (xb_ext[:, dy * WL:dy * WL + CH] if src is None
                  else src[:, dy * WL:dy * WL + CH])
            term = dww_ref[:, 3 * dy + dx:3 * dy + dx + 1] * sl
            y = term if y is None else y + term
    p_avg = jnp.mean(ph_ref[0], axis=0, keepdims=True)
    q = jnp.dot(p_avg, aw_ref[...], preferred_element_type=jnp.float32)
    fi_s = jnp.dot(q, xc_ref[0], preferred_element_type=jnp.float32)
    fx = (y + al_ref[...] * fi_s) * _interior_mask(jj * CH, CH, jnp.float32)
    fx_ref[0] = fx.astype(jnp.bfloat16)
    part = jnp.sum(fx, axis=1, keepdims=True)

    @pl.when(jj == 0)
    def _():
        gap_ref[...] = part

    @pl.when(jj > 0)
    def _():
        gap_ref[...] = gap_ref[...] + part

    @pl.when(jj == NCJ - 1)
    def _():
        gap = gap_ref[...] * (1.0 / (H * W))
        scores = (jnp.dot(rw_ref[...], gap, preferred_element_type=jnp.float32)
                  + rb_ref[...])
        sc_ref[pl.ds(E * b, E), :] = scores


@functools.partial(
    pl.kernel,
    out_type=[jax.ShapeDtypeStruct((2 * E,), jnp.int32),
              jax.ShapeDtypeStruct((2 * E,), jnp.float32)],
    mesh=plsc.VectorSubcoreMesh(core_axis_name="c", subcore_axis_name="s"),
    scratch_types=[pltpu.VMEM((2 * E,), jnp.float32),
                   pltpu.VMEM((2 * E,), jnp.int32),
                   pltpu.VMEM((2 * E,), jnp.float32)],
    compiler_params=pltpu.CompilerParams(needs_layout_passes=False),
)
def _route_sc(sc_hbm, idx_hbm, gate_hbm, sv_ref, iv_ref, gv_ref):
    wid = jax.lax.axis_index("s") * 2 + jax.lax.axis_index("c")

    @pl.when(wid == 0)
    def _():
        pltpu.sync_copy(sc_hbm, sv_ref)
        sv = sv_ref[...]
        iota = jax.lax.iota(jnp.int32, 2 * E)
        neg = jnp.float32(-jnp.inf)
        for bb in range(2):
            half = (iota >= E * bb) & (iota < E * bb + E)
            kb = jnp.where(half, sv, neg)
            skeys, svals = plsc.sort_key_val(kb, iota & (E - 1),
                                             descending=True)
            gv = 1.0 / (1.0 + jnp.exp(plsc.cummax(skeys) - skeys))
            iv_ref[...] = svals
            gv_ref[...] = gv
            pltpu.sync_copy(iv_ref.at[pl.ds(0, E)],
                            idx_hbm.at[pl.ds(E * bb, E)])
            pltpu.sync_copy(gv_ref.at[pl.ds(0, E)],
                            gate_hbm.at[pl.ds(E * bb, E)])


def _moe_kernel(idx_ref, gate_ref, fxm_ref, fxc_ref, fxp_ref, xres_ref,
                w1a_ref, w1b_ref, w2a_ref, w2b_ref, out_ref):
    b = pl.program_id(0)
    jj = pl.program_id(1)
    HL = CH + 2 * HALO
    ext = jnp.concatenate(
        [fxm_ref[0, :, CH - (2 * HALO - WL):], fxc_ref[0],
         fxp_ref[0, :, :2 * HALO + WL]], axis=1)
    lane = jax.lax.broadcasted_iota(jnp.int32, (1, CH + 4 * HALO), 1)
    ext = jnp.where((jj == 0) & (lane == HALO), jnp.zeros((), jnp.bfloat16), ext)
    g1 = gate_ref[E * b + 1]
    g0 = 1.0 - g1
    w1pa = w1a_ref[0]
    w1pb = w1b_ref[0]
    w2pa = (w2a_ref[0].astype(jnp.float32) * g0).astype(jnp.bfloat16)
    w2pb = (w2b_ref[0].astype(jnp.float32) * g1).astype(jnp.bfloat16)
    SE = CH + 4 * HALO - 2
    stacked = jnp.concatenate(
        [ext[:, 0:SE], ext[:, 1:SE + 1], ext[:, 2:SE + 2]], axis=0)
    acc = None
    for dy in range(3):
        wdy = jnp.concatenate(
            [w1pa[:, 3 * C * dy:3 * C * (dy + 1)],
             w1pb[:, 3 * C * dy:3 * C * (dy + 1)]], axis=0)
        term = jnp.dot(wdy, stacked[:, dy * WL:dy * WL + HL],
                       preferred_element_type=jnp.float32)
        acc = term if acc is None else acc + term
    gelu = acc * 0.5 * (1.0 + jax.lax.erf(acc * (2.0 ** -0.5)))
    hmask = _interior_mask(jj * CH - 1, HL, jnp.float32)
    hmid = (gelu * hmask).astype(jnp.bfloat16)
    acc2 = None
    for t, s in enumerate(SHIFTS):
        w2cat = jnp.concatenate(
            [w2pa[:, C * t:C * (t + 1)], w2pb[:, C * t:C * (t + 1)]],
            axis=1)
        term = jnp.dot(w2cat, hmid[:, s:s + CH],
                       preferred_element_type=jnp.float32)
        acc2 = term if acc2 is None else acc2 + term
    out_ref[0] = acc2.reshape(C, RB, WL)[:, :, :W] + xres_ref[0]


def kernel(x, P_hat, proj_a_w, proj_b_w, dw_b_w, fi_align_w, router_w, router_b,
           expert_w1, expert_w2):
    B = x.shape[0]
    aw = proj_a_w.reshape(C, C)
    bw = proj_b_w.reshape(C, C)
    dww = dw_b_w.reshape(C, 9)
    al = fi_align_w.reshape(C, 1)
    rb = router_b.reshape(E, 1)
    x_flat = jnp.pad(x, ((0, 0), (0, 0), (1, LTOT // WL - 1 - H),
                         (0, WL - W))).reshape(B, C, LTOT)

    blk = lambda f: pl.BlockSpec((1, C, CH), f)
    fx, scores = pl.pallas_call(
        _pre_kernel,
        grid=(B, NCJ),
        in_specs=[
            pl.BlockSpec((1, P_hat.shape[1], C), lambda b, j: (b, 0, 0)),
            blk(lambda b, j: (b, 0, jnp.maximum(j - 1, 0))),
            blk(lambda b, j: (b, 0, j)),
            blk(lambda b, j: (b, 0, jnp.minimum(j + 1, NCJ - 1))),
            pl.BlockSpec((C, C), lambda b, j: (0, 0)),
            pl.BlockSpec((C, C), lambda b, j: (0, 0)),
            pl.BlockSpec((C, 9), lambda b, j: (0, 0)),
            pl.BlockSpec((C, 1), lambda b, j: (0, 0)),
            pl.BlockSpec((E, C), lambda b, j: (0, 0)),
            pl.BlockSpec((E, 1), lambda b, j: (0, 0)),
        ],
        out_specs=[
            blk(lambda b, j: (b, 0, j)),
            pl.BlockSpec((2 * E, 1), lambda b, j: (0, 0)),
        ],
        out_shape=[
            jax.ShapeDtypeStruct((B, C, LTOT), jnp.bfloat16),
            jax.ShapeDtypeStruct((2 * E, 1), jnp.float32),
        ],
        scratch_shapes=[
            pltpu.VMEM((C, 1), jnp.float32),
            pltpu.VMEM((C, CH + 2 * WL), jnp.float32),
            pltpu.VMEM((C, CH + 2 * WL), jnp.float32),
        ],
        compiler_params=pltpu.CompilerParams(
            dimension_semantics=("arbitrary", "arbitrary"),
            allow_input_fusion=[False, True, True, True, False, False,
                                False, False, False, False]),
    )(P_hat, x_flat, x_flat, x_flat, aw, bw, dww, al, router_w, rb)
    idx, gate = _route_sc(scores.reshape(2 * E))

    w1v = (expert_w1.transpose(0, 1, 3, 4, 2).reshape(E, C, 9 * C)
           .astype(jnp.bfloat16))
    w2v = (expert_w2.transpose(0, 1, 3, 4, 2).reshape(E, C, 9 * C)
           .astype(jnp.bfloat16))

    wblk = lambda f: pl.BlockSpec((1, C, 9 * C), f)
    grid_spec = pltpu.PrefetchScalarGridSpec(
        num_scalar_prefetch=2,
        grid=(B, NCO),
        in_specs=[
            blk(lambda b, j, idx, gate: (b, 0, jnp.maximum(j - 1, 0))),
            blk(lambda b, j, idx, gate: (b, 0, j)),
            blk(lambda b, j, idx, gate: (b, 0, jnp.minimum(j + 1, NCJ - 1))),
            pl.BlockSpec((1, C, RB, W), lambda b, j, idx, gate: (b, 0, j, 0)),
            wblk(lambda b, j, idx, gate: (idx[E * b], 0, 0)),
            wblk(lambda b, j, idx, gate: (idx[E * b + 1], 0, 0)),
            wblk(lambda b, j, idx, gate: (idx[E * b], 0, 0)),
            wblk(lambda b, j, idx, gate: (idx[E * b + 1], 0, 0)),
        ],
        out_specs=pl.BlockSpec((1, C, RB, W),
                               lambda b, j, idx, gate: (b, 0, j, 0)),
    )
    out = pl.pallas_call(
        _moe_kernel,
        grid_spec=grid_spec,
        out_shape=jax.ShapeDtypeStruct((B, C, H, W), jnp.float32),
        compiler_params=pltpu.CompilerParams(
            dimension_semantics=("parallel", "parallel")),
    )(idx, gate, fx, fx, fx, x, w1v, w1v, w2v, w2v)
    return out


# speedup vs baseline: 1.0075x; 1.0075x over previous
import functools

import jax
import jax.numpy as jnp
from jax.experimental import pallas as pl
from jax.experimental.pallas import tpu as pltpu
from jax.experimental.pallas import tpu_sc as plsc

C = 96
H = W = 224
WL = 256
CH = 8192
NCJ = 8
LTOT = CH * NCJ
RB = 32
NCO = 7
E = 8
TOPK = 2
HALO = WL + 1
SHIFTS = tuple(dy * WL + dx for dy in range(3) for dx in range(3))


def _interior_mask(start, length, dtype):
    g = start + jax.lax.broadcasted_iota(jnp.int32, (1, length), 1)
    g = jnp.maximum(g, 0)
    row = g // WL
    col = g - row * WL
    return ((row >= 1) & (row <= H) & (col < W)).astype(dtype)


def _pre_kernel(ph_ref, xm_ref, xc_ref, xp_ref, aw_ref, bw_ref, dww_ref, al_ref,
                rw_ref, rb_ref, fx_ref, sc_ref,
                gap_ref, r1_ref, r2_ref):
    b = pl.program_id(0)
    jj = pl.program_id(1)
    lane = jax.lax.broadcasted_iota(jnp.int32, (1, HALO), 1)
    left = jnp.where((jj == 0) & (lane == HALO - 1), 0.0,
                     xm_ref[0, :, CH - HALO:])
    ext = jnp.concatenate([left, xc_ref[0], xp_ref[0, :, :HALO]], axis=1)
    xb_ext = jnp.dot(bw_ref[...], ext, preferred_element_type=jnp.float32)
    r1_ref[...] = xb_ext[:, 1:1 + CH + 2 * WL]
    r2_ref[...] = xb_ext[:, 2:2 + CH + 2 * WL]
    y = None
    for dy in range(3):
        for dx, src in ((0, None), (1, r1_ref), (2, r2_ref)):
            sl = (xb_ext[:, dy * WL:dy * WL + CH] if src is None
                  else src[:, dy * WL:dy * WL + CH])
            term = dww_ref[:, 3 * dy + dx:3 * dy + dx + 1] * sl
            y = term if y is None else y + term
    p_avg = jnp.mean(ph_ref[0], axis=0, keepdims=True)
    q = jnp.dot(p_avg, aw_ref[...], preferred_element_type=jnp.float32)
    fi_s = jnp.dot(q, xc_ref[0], preferred_element_type=jnp.float32)
    fx = (y + al_ref[...] * fi_s) * _interior_mask(jj * CH, CH, jnp.float32)
    fx_ref[0] = fx.astype(jnp.bfloat16)
    part = jnp.sum(fx, axis=1, keepdims=True)

    @pl.when(jj == 0)
    def _():
        gap_ref[...] = part

    @pl.when(jj > 0)
    def _():
        gap_ref[...] = gap_ref[...] + part

    @pl.when(jj == NCJ - 1)
    def _():
        gap = gap_ref[...] * (1.0 / (H * W))
        scores = (jnp.dot(rw_ref[...], gap, preferred_element_type=jnp.float32)
                  + rb_ref[...])
        sc_ref[pl.ds(E * b, E), :] = scores


@functools.partial(
    pl.kernel,
    out_type=[jax.ShapeDtypeStruct((2 * E,), jnp.int32),
              jax.ShapeDtypeStruct((2 * E,), jnp.float32)],
    mesh=plsc.VectorSubcoreMesh(core_axis_name="c", subcore_axis_name="s"),
    scratch_types=[pltpu.VMEM((2 * E,), jnp.float32),
                   pltpu.VMEM((2 * E,), jnp.int32),
                   pltpu.VMEM((2 * E,), jnp.float32)],
    compiler_params=pltpu.CompilerParams(needs_layout_passes=False),
)
def _route_sc(sc_hbm, idx_hbm, gate_hbm, sv_ref, iv_ref, gv_ref):
    wid = jax.lax.axis_index("s") * 2 + jax.lax.axis_index("c")

    @pl.when(wid == 0)
    def _():
        pltpu.sync_copy(sc_hbm, sv_ref)
        sv = sv_ref[...]
        iota = jax.lax.iota(jnp.int32, 2 * E)
        neg = jnp.float32(-jnp.inf)
        for bb in range(2):
            half = (iota >= E * bb) & (iota < E * bb + E)
            kb = jnp.where(half, sv, neg)
            skeys, svals = plsc.sort_key_val(kb, iota & (E - 1),
                                             descending=True)
            gv = 1.0 / (1.0 + jnp.exp(plsc.cummax(skeys) - skeys))
            iv_ref[...] = svals
            gv_ref[...] = gv
            pltpu.sync_copy(iv_ref.at[pl.ds(0, E)],
                            idx_hbm.at[pl.ds(E * bb, E)])
            pltpu.sync_copy(gv_ref.at[pl.ds(0, E)],
                            gate_hbm.at[pl.ds(E * bb, E)])


def _moe_kernel(idx_ref, gate_ref, fxm_ref, fxc_ref, fxp_ref, xres_ref,
                w1a_ref, w1b_ref, w2a_ref, w2b_ref, out_ref):
    b = pl.program_id(0)
    jj = pl.program_id(1)
    HL = CH + 2 * HALO
    lane = jax.lax.broadcasted_iota(jnp.int32, (1, 2 * HALO - WL), 1)
    left = jnp.where((jj == 0) & (lane == 2 * HALO - WL - 1),
                     jnp.zeros((), jnp.bfloat16),
                     fxm_ref[0, :, CH - (2 * HALO - WL):])
    ext = jnp.concatenate(
        [left, fxc_ref[0],
         fxp_ref[0, :, :2 * HALO + WL]], axis=1)
    g1 = gate_ref[E * b + 1]
    g0 = 1.0 - g1
    w1pa = w1a_ref[0]
    w1pb = w1b_ref[0]
    w2pa = (w2a_ref[0].astype(jnp.float32) * g0).astype(jnp.bfloat16)
    w2pb = (w2b_ref[0].astype(jnp.float32) * g1).astype(jnp.bfloat16)
    SE = CH + 4 * HALO - 2
    stacked = jnp.concatenate(
        [ext[:, 0:SE], ext[:, 1:SE + 1], ext[:, 2:SE + 2]], axis=0)
    acc = None
    for dy in range(3):
        wdy = jnp.concatenate(
            [w1pa[:, 3 * C * dy:3 * C * (dy + 1)],
             w1pb[:, 3 * C * dy:3 * C * (dy + 1)]], axis=0)
        term = jnp.dot(wdy, stacked[:, dy * WL:dy * WL + HL],
                       preferred_element_type=jnp.float32)
        acc = term if acc is None else acc + term
    gelu = acc * 0.5 * (1.0 + jax.lax.erf(acc * (2.0 ** -0.5)))
    hmask = _interior_mask(jj * CH - 1, HL, jnp.float32)
    hmid = (gelu * hmask).astype(jnp.bfloat16)
    acc2 = None
    for t, s in enumerate(SHIFTS):
        w2cat = jnp.concatenate(
            [w2pa[:, C * t:C * (t + 1)], w2pb[:, C * t:C * (t + 1)]],
            axis=1)
        term = jnp.dot(w2cat, hmid[:, s:s + CH],
                       preferred_element_type=jnp.float32)
        acc2 = term if acc2 is None else acc2 + term
    out_ref[0] = acc2.reshape(C, RB, WL)[:, :, :W] + xres_ref[0]


def kernel(x, P_hat, proj_a_w, proj_b_w, dw_b_w, fi_align_w, router_w, router_b,
           expert_w1, expert_w2):
    B = x.shape[0]
    aw = proj_a_w.reshape(C, C)
    bw = proj_b_w.reshape(C, C)
    dww = dw_b_w.reshape(C, 9)
    al = fi_align_w.reshape(C, 1)
    rb = router_b.reshape(E, 1)
    x_flat = jnp.pad(x, ((0, 0), (0, 0), (1, LTOT // WL - 1 - H),
                         (0, WL - W))).reshape(B, C, LTOT)

    blk = lambda f: pl.BlockSpec((1, C, CH), f)
    fx, scores = pl.pallas_call(
        _pre_kernel,
        grid=(B, NCJ),
        in_specs=[
            pl.BlockSpec((1, P_hat.shape[1], C), lambda b, j: (b, 0, 0)),
            blk(lambda b, j: (b, 0, jnp.maximum(j - 1, 0))),
            blk(lambda b, j: (b, 0, j)),
            blk(lambda b, j: (b, 0, jnp.minimum(j + 1, NCJ - 1))),
            pl.BlockSpec((C, C), lambda b, j: (0, 0)),
            pl.BlockSpec((C, C), lambda b, j: (0, 0)),
            pl.BlockSpec((C, 9), lambda b, j: (0, 0)),
            pl.BlockSpec((C, 1), lambda b, j: (0, 0)),
            pl.BlockSpec((E, C), lambda b, j: (0, 0)),
            pl.BlockSpec((E, 1), lambda b, j: (0, 0)),
        ],
        out_specs=[
            blk(lambda b, j: (b, 0, j)),
            pl.BlockSpec((2 * E, 1), lambda b, j: (0, 0)),
        ],
        out_shape=[
            jax.ShapeDtypeStruct((B, C, LTOT), jnp.bfloat16),
            jax.ShapeDtypeStruct((2 * E, 1), jnp.float32),
        ],
        scratch_shapes=[
            pltpu.VMEM((C, 1), jnp.float32),
            pltpu.VMEM((C, CH + 2 * WL), jnp.float32),
            pltpu.VMEM((C, CH + 2 * WL), jnp.float32),
        ],
        compiler_params=pltpu.CompilerParams(
            dimension_semantics=("arbitrary", "arbitrary")),
    )(P_hat, x_flat, x_flat, x_flat, aw, bw, dww, al, router_w, rb)
    idx, gate = _route_sc(scores.reshape(2 * E))

    w1v = (expert_w1.transpose(0, 1, 3, 4, 2).reshape(E, C, 9 * C)
           .astype(jnp.bfloat16))
    w2v = (expert_w2.transpose(0, 1, 3, 4, 2).reshape(E, C, 9 * C)
           .astype(jnp.bfloat16))

    wblk = lambda f: pl.BlockSpec((1, C, 9 * C), f)
    grid_spec = pltpu.PrefetchScalarGridSpec(
        num_scalar_prefetch=2,
        grid=(B, NCO),
        in_specs=[
            blk(lambda b, j, idx, gate: (b, 0, jnp.maximum(j - 1, 0))),
            blk(lambda b, j, idx, gate: (b, 0, j)),
            blk(lambda b, j, idx, gate: (b, 0, jnp.minimum(j + 1, NCJ - 1))),
            pl.BlockSpec((1, C, RB, W), lambda b, j, idx, gate: (b, 0, j, 0)),
            wblk(lambda b, j, idx, gate: (idx[E * b], 0, 0)),
            wblk(lambda b, j, idx, gate: (idx[E * b + 1], 0, 0)),
            wblk(lambda b, j, idx, gate: (idx[E * b], 0, 0)),
            wblk(lambda b, j, idx, gate: (idx[E * b + 1], 0, 0)),
        ],
        out_specs=pl.BlockSpec((1, C, RB, W),
                               lambda b, j, idx, gate: (b, 0, j, 0)),
    )
    out = pl.pallas_call(
        _moe_kernel,
        grid_spec=grid_spec,
        out_shape=jax.ShapeDtypeStruct((B, C, H, W), jnp.float32),
        compiler_params=pltpu.CompilerParams(
            dimension_semantics=("parallel", "parallel")),
    )(idx, gate, fx, fx, fx, x, w1v, w1v, w2v, w2v)
    return out


# speedup vs baseline: 1.0163x; 1.0087x over previous
import functools

import jax
import jax.numpy as jnp
from jax.experimental import pallas as pl
from jax.experimental.pallas import tpu as pltpu
from jax.experimental.pallas import tpu_sc as plsc

C = 96
H = W = 224
WL = 256
CH = 8192
NCJ = 8
LTOT = CH * NCJ
RB = 32
NCO = 7
E = 8
HALO = WL + 1
SHIFTS = tuple(dy * WL + dx for dy in range(3) for dx in range(3))


def _interior_mask(start, length, dtype):
    g = start + jax.lax.broadcasted_iota(jnp.int32, (1, length), 1)
    g = jnp.maximum(g, 0)
    row = g // WL
    col = g - row * WL
    return ((row >= 1) & (row <= H) & (col < W)).astype(dtype)


def _pre_kernel(ph_ref, xm_ref, xc_ref, xp_ref, aw_ref, bw_ref, dww_ref, al_ref,
                rw_ref, rb_ref, fx_ref, sc_ref,
                gap_ref, r1_ref, r2_ref):
    b = pl.program_id(0)
    jj = pl.program_id(1)
    lane = jax.lax.broadcasted_iota(jnp.int32, (1, HALO), 1)
    left = jnp.where((jj == 0) & (lane == HALO - 1), 0.0,
                     xm_ref[0, :, CH - HALO:])
    ext = jnp.concatenate([left, xc_ref[0], xp_ref[0, :, :HALO]], axis=1)
    xb_ext = jnp.dot(bw_ref[...], ext, preferred_element_type=jnp.float32)
    r1_ref[...] = xb_ext[:, 1:1 + CH + 2 * WL]
    r2_ref[...] = xb_ext[:, 2:2 + CH + 2 * WL]
    y = None
    for dy in range(3):
        for dx, src in ((0, None), (1, r1_ref), (2, r2_ref)):
            sl = (xb_ext[:, dy * WL:dy * WL + CH] if src is None
                  else src[:, dy * WL:dy * WL + CH])
            term = dww_ref[:, 3 * dy + dx:3 * dy + dx + 1] * sl
            y = term if y is None else y + term
    p_avg = jnp.mean(ph_ref[0], axis=0, keepdims=True)
    q = jnp.dot(p_avg, aw_ref[...], preferred_element_type=jnp.float32)
    fi_s = jnp.dot(q, xc_ref[0], preferred_element_type=jnp.float32)
    fx = (y + al_ref[...] * fi_s) * _interior_mask(jj * CH, CH, jnp.float32)
    fx_ref[0] = fx.astype(jnp.bfloat16)
    part = jnp.sum(fx, axis=1, keepdims=True)

    @pl.when(jj == 0)
    def _():
        gap_ref[...] = part

    @pl.when(jj > 0)
    def _():
        gap_ref[...] = gap_ref[...] + part

    @pl.when(jj == NCJ - 1)
    def _():
        gap = gap_ref[...] * (1.0 / (H * W))
        scores = (jnp.dot(rw_ref[...], gap, preferred_element_type=jnp.float32)
                  + rb_ref[...])
        sc_ref[pl.ds(E * b, E), :] = scores


@functools.partial(
    pl.kernel,
    out_type=[jax.ShapeDtypeStruct((2 * E,), jnp.int32),
              jax.ShapeDtypeStruct((2 * E,), jnp.float32)],
    mesh=plsc.VectorSubcoreMesh(core_axis_name="c", subcore_axis_name="s"),
    scratch_types=[pltpu.VMEM((2 * E,), jnp.float32),
                   pltpu.VMEM((2 * E,), jnp.int32),
                   pltpu.VMEM((2 * E,), jnp.float32)],
    compiler_params=pltpu.CompilerParams(needs_layout_passes=False),
)
def _route_sc(sc_hbm, idx_hbm, gate_hbm, sv_ref, iv_ref, gv_ref):
    wid = jax.lax.axis_index("s") * 2 + jax.lax.axis_index("c")

    @pl.when(wid == 0)
    def _():
        pltpu.sync_copy(sc_hbm, sv_ref)
        sv = sv_ref[...]
        iota = jax.lax.iota(jnp.int32, 2 * E)
        neg = jnp.float32(-jnp.inf)
        for bb in range(2):
            half = (iota >= E * bb) & (iota < E * bb + E)
            kb = jnp.where(half, sv, neg)
            skeys, svals = plsc.sort_key_val(kb, iota & (E - 1),
                                             descending=True)
            gv = 1.0 / (1.0 + jnp.exp(plsc.cummax(skeys) - skeys))
            iv_ref[...] = svals
            gv_ref[...] = gv
            pltpu.sync_copy(iv_ref.at[pl.ds(0, E)],
                            idx_hbm.at[pl.ds(E * bb, E)])
            pltpu.sync_copy(gv_ref.at[pl.ds(0, E)],
                            gate_hbm.at[pl.ds(E * bb, E)])


def _moe_kernel(idx_ref, gate_ref, fxm_ref, fxc_ref, fxp_ref, xres_ref,
                w1a_ref, w1b_ref, w2a_ref, w2b_ref, out_ref):
    b = pl.program_id(0)
    jj = pl.program_id(1)
    HL = CH + 2 * HALO
    SE = CH + 4 * HALO - 2

    def _row(sh):
        lw = 2 * HALO - WL - sh
        lane = jax.lax.broadcasted_iota(jnp.int32, (1, lw), 1)
        left = jnp.where((jj == 0) & (lane == lw - 1),
                         jnp.zeros((), jnp.bfloat16),
                         fxm_ref[0, :, CH - lw:])
        return jnp.concatenate(
            [left, fxc_ref[0], fxp_ref[0, :, :SE - lw - CH]], axis=1)
    g1 = gate_ref[E * b + 1]
    g0 = 1.0 - g1
    w1pa = w1a_ref[0]
    w1pb = w1b_ref[0]
    w2pa = (w2a_ref[0].astype(jnp.float32) * g0).astype(jnp.bfloat16)
    w2pb = (w2b_ref[0].astype(jnp.float32) * g1).astype(jnp.bfloat16)
    stacked = jnp.concatenate([_row(0), _row(1), _row(2)], axis=0)
    acc = None
    for dy in range(3):
        wdy = jnp.concatenate(
            [w1pa[:, 3 * C * dy:3 * C * (dy + 1)],
             w1pb[:, 3 * C * dy:3 * C * (dy + 1)]], axis=0)
        term = jnp.dot(wdy, stacked[:, dy * WL:dy * WL + HL],
                       preferred_element_type=jnp.float32)
        acc = term if acc is None else acc + term
    gelu = acc * 0.5 * (1.0 + jax.lax.erf(acc * (2.0 ** -0.5)))
    hmask = _interior_mask(jj * CH - 1, HL, jnp.float32)
    hmid = (gelu * hmask).astype(jnp.bfloat16)
    acc2 = None
    for t, s in enumerate(SHIFTS):
        w2cat = jnp.concatenate(
            [w2pa[:, C * t:C * (t + 1)], w2pb[:, C * t:C * (t + 1)]],
            axis=1)
        term = jnp.dot(w2cat, hmid[:, s:s + CH],
                       preferred_element_type=jnp.float32)
        acc2 = term if acc2 is None else acc2 + term
    out_ref[0] = acc2.reshape(C, RB, WL)[:, :, :W] + xres_ref[0]


def kernel(x, P_hat, proj_a_w, proj_b_w, dw_b_w, fi_align_w, router_w, router_b,
           expert_w1, expert_w2):
    B = x.shape[0]
    aw = proj_a_w.reshape(C, C)
    bw = proj_b_w.reshape(C, C)
    dww = dw_b_w.reshape(C, 9)
    al = fi_align_w.reshape(C, 1)
    rb = router_b.reshape(E, 1)
    x_flat = jnp.pad(x, ((0, 0), (0, 0), (1, LTOT // WL - 1 - H),
                         (0, WL - W))).reshape(B, C, LTOT)

    blk = lambda f: pl.BlockSpec((1, C, CH), f)
    fx, scores = pl.pallas_call(
        _pre_kernel,
        grid=(B, NCJ),
        in_specs=[
            pl.BlockSpec((1, P_hat.shape[1], C), lambda b, j: (b, 0, 0)),
            blk(lambda b, j: (b, 0, jnp.maximum(j - 1, 0))),
            blk(lambda b, j: (b, 0, j)),
            blk(lambda b, j: (b, 0, jnp.minimum(j + 1, NCJ - 1))),
            pl.BlockSpec((C, C), lambda b, j: (0, 0)),
            pl.BlockSpec((C, C), lambda b, j: (0, 0)),
            pl.BlockSpec((C, 9), lambda b, j: (0, 0)),
            pl.BlockSpec((C, 1), lambda b, j: (0, 0)),
            pl.BlockSpec((E, C), lambda b, j: (0, 0)),
            pl.BlockSpec((E, 1), lambda b, j: (0, 0)),
        ],
        out_specs=[
            blk(lambda b, j: (b, 0, j)),
            pl.BlockSpec((2 * E, 1), lambda b, j: (0, 0)),
        ],
        out_shape=[
            jax.ShapeDtypeStruct((B, C, LTOT), jnp.bfloat16),
            jax.ShapeDtypeStruct((2 * E, 1), jnp.float32),
        ],
        scratch_shapes=[
            pltpu.VMEM((C, 1), jnp.float32),
            pltpu.VMEM((C, CH + 2 * WL), jnp.float32),
            pltpu.VMEM((C, CH + 2 * WL), jnp.float32),
        ],
        compiler_params=pltpu.CompilerParams(
            dimension_semantics=("arbitrary", "arbitrary")),
    )(P_hat, x_flat, x_flat, x_flat, aw, bw, dww, al, router_w, rb)
    idx, gate = _route_sc(scores.reshape(2 * E))

    w1v = (expert_w1.transpose(0, 1, 3, 4, 2).reshape(E, C, 9 * C)
           .astype(jnp.bfloat16))
    w2v = (expert_w2.transpose(0, 1, 3, 4, 2).reshape(E, C, 9 * C)
           .astype(jnp.bfloat16))

    wblk = lambda f: pl.BlockSpec((1, C, 9 * C), f)
    grid_spec = pltpu.PrefetchScalarGridSpec(
        num_scalar_prefetch=2,
        grid=(B, NCO),
        in_specs=[
            blk(lambda b, j, idx, gate: (b, 0, jnp.maximum(j - 1, 0))),
            blk(lambda b, j, idx, gate: (b, 0, j)),
            blk(lambda b, j, idx, gate: (b, 0, jnp.minimum(j + 1, NCJ - 1))),
            pl.BlockSpec((1, C, RB, W), lambda b, j, idx, gate: (b, 0, j, 0)),
            wblk(lambda b, j, idx, gate: (idx[E * b], 0, 0)),
            wblk(lambda b, j, idx, gate: (idx[E * b + 1], 0, 0)),
            wblk(lambda b, j, idx, gate: (idx[E * b], 0, 0)),
            wblk(lambda b, j, idx, gate: (idx[E * b + 1], 0, 0)),
        ],
        out_specs=pl.BlockSpec((1, C, RB, W),
                               lambda b, j, idx, gate: (b, 0, j, 0)),
    )
    out = pl.pallas_call(
        _moe_kernel,
        grid_spec=grid_spec,
        out_shape=jax.ShapeDtypeStruct((B, C, H, W), jnp.float32),
        compiler_params=pltpu.CompilerParams(
            dimension_semantics=("parallel", "parallel")),
    )(idx, gate, fx, fx, fx, x, w1v, w1v, w2v, w2v)
    return out
